# Initial kernel scaffold; baseline (speedup 1.0000x reference)
import jax
import jax.numpy as jnp
from jax.experimental import pallas as pl


def kernel(queries, keys):
    raise NotImplementedError("write your pallas kernel here")


# speedup vs baseline: 1.6494x; 1.6494x over previous
import functools

import jax
import jax.numpy as jnp
from jax.experimental import pallas as pl
from jax.experimental.pallas import tpu as pltpu

TOPK = 20
KB = 2048

_INT_MAX = 2147483647


def _extract_topk(v, idx, k):
    vs, ids = [], []
    for _ in range(k):
        m = jnp.max(v, axis=1, keepdims=True)
        pick = jnp.min(jnp.where(v == m, idx, _INT_MAX), axis=1, keepdims=True)
        vs.append(m)
        ids.append(pick)
        v = jnp.where(idx == pick, -jnp.inf, v)
    return jnp.concatenate(vs, axis=1), jnp.concatenate(ids, axis=1)


def _body(nkeys, nkb, q_ref, k_ref, pv_ref, pi_ref, vals_scr, idx_scr):
    j = pl.program_id(0)

    @pl.when(j == 0)
    def _init():
        vals_scr[...] = jnp.full(vals_scr.shape, -jnp.inf, jnp.float32)
        idx_scr[...] = jnp.zeros(idx_scr.shape, jnp.int32)

    s = jax.lax.dot_general(
        q_ref[...], k_ref[...], (((1,), (1,)), ((), ())),
        preferred_element_type=jnp.float32)
    col = jax.lax.broadcasted_iota(jnp.int32, s.shape, 1) + j * KB
    s = jnp.where(col < nkeys, s, -jnp.inf)
    tv, ti = _extract_topk(s, col, TOPK)
    mv = jnp.concatenate([vals_scr[...], tv], axis=1)
    mi = jnp.concatenate([idx_scr[...], ti], axis=1)
    fv, fi = _extract_topk(mv, mi, TOPK)
    vals_scr[...] = fv
    idx_scr[...] = fi

    @pl.when(j == nkb - 1)
    def _finalize():
        e = jnp.exp(fv - fv[:, :1])
        pv_ref[...] = e / jnp.sum(e, axis=1, keepdims=True)
        pi_ref[...] = fi


def kernel(queries, keys):
    q, d = queries.shape
    n, _ = keys.shape
    nkb = pl.cdiv(n, KB)
    prior, idx = pl.pallas_call(
        functools.partial(_body, n, nkb),
        grid=(nkb,),
        in_specs=[
            pl.BlockSpec((q, d), lambda j: (0, 0)),
            pl.BlockSpec((KB, d), lambda j: (j, 0)),
        ],
        out_specs=[
            pl.BlockSpec((q, TOPK), lambda j: (0, 0)),
            pl.BlockSpec((q, TOPK), lambda j: (0, 0)),
        ],
        out_shape=[
            jax.ShapeDtypeStruct((q, TOPK), jnp.float32),
            jax.ShapeDtypeStruct((q, TOPK), jnp.int32),
        ],
        scratch_shapes=[
            pltpu.VMEM((q, TOPK), jnp.float32),
            pltpu.VMEM((q, TOPK), jnp.int32),
        ],
    )(queries, keys)
    return prior, idx


# speedup vs baseline: 3.9828x; 2.4147x over previous
import functools

import jax
import jax.numpy as jnp
from jax.experimental import pallas as pl
from jax.experimental.pallas import tpu as pltpu

TOPK = 20
KB = 2048
NSL = KB // 128
RPG = 4

_INT_MAX = 2147483647


def _extract_topk(v, idx, k):
    vs, ids = [], []
    for _ in range(k):
        m = jnp.max(v, axis=1, keepdims=True)
        pick = jnp.min(jnp.where(v == m, idx, _INT_MAX), axis=1, keepdims=True)
        vs.append(m)
        ids.append(pick)
        v = jnp.where(idx == pick, -jnp.inf, v)
    return jnp.concatenate(vs, axis=1), jnp.concatenate(ids, axis=1)


def _body_fast(nkeys, nkb, q_ref, k_ref, pv_ref, pi_ref, fl_ref,
               rv_scr, ri_scr, v4_scr):
    j = pl.program_id(0)

    @pl.when(j == 0)
    def _init():
        rv_scr[...] = jnp.full(rv_scr.shape, -jnp.inf, jnp.float32)
        ri_scr[...] = jnp.full(ri_scr.shape, -1, jnp.int32)

    s = jax.lax.dot_general(
        q_ref[...], k_ref[...], (((1,), (1,)), ((), ())),
        preferred_element_type=jnp.float32)
    col = jax.lax.broadcasted_iota(jnp.int32, s.shape, 1) + j * KB
    s = jnp.where(col < nkeys, s, -jnp.inf)

    cur = [s[:, a * 128:(a + 1) * 128] for a in range(NSL)]
    lane = jax.lax.broadcasted_iota(jnp.int32, cur[0].shape, 1)
    vs, gs = [], []
    for _ in range(RPG):
        m = cur[0]
        for a in range(1, NSL):
            m = jnp.maximum(m, cur[a])
        amin = jnp.full(m.shape, NSL, jnp.int32)
        for a in range(NSL):
            amin = jnp.minimum(amin, jnp.where(cur[a] == m, a, NSL))
        for a in range(NSL):
            cur[a] = jnp.where(amin == a, -jnp.inf, cur[a])
        vs.append(m)
        gs.append(j * KB + amin * 128 + lane)
    v4_scr[:, pl.ds(j * 128, 128)] = vs[RPG - 1]

    mv = jnp.concatenate([rv_scr[...]] + vs, axis=1)
    mi = jnp.concatenate([ri_scr[...]] + gs, axis=1)
    fv, fi = _extract_topk(mv, mi, TOPK)
    pad = rv_scr.shape[1] - TOPK
    rv_scr[...] = jnp.concatenate(
        [fv, jnp.full((fv.shape[0], pad), -jnp.inf, jnp.float32)], axis=1)
    ri_scr[...] = jnp.concatenate(
        [fi, jnp.full((fi.shape[0], pad), -1, jnp.int32)], axis=1)

    @pl.when(j == nkb - 1)
    def _finalize():
        e = jnp.exp(fv - fv[:, :1])
        pv_ref[...] = e / jnp.sum(e, axis=1, keepdims=True)
        pi_ref[...] = fi
        v20 = fv[:, TOPK - 1:TOPK]
        bad = jnp.any(v4_scr[...] >= v20, axis=1, keepdims=True)
        fl_ref[...] = jnp.broadcast_to(bad, fl_ref.shape).astype(jnp.int32)


def _fast(queries, keys):
    q, d = queries.shape
    n, _ = keys.shape
    nkb = pl.cdiv(n, KB)
    return pl.pallas_call(
        functools.partial(_body_fast, n, nkb),
        grid=(nkb,),
        in_specs=[
            pl.BlockSpec((q, d), lambda j: (0, 0)),
            pl.BlockSpec((KB, d), lambda j: (j, 0)),
        ],
        out_specs=[
            pl.BlockSpec((q, TOPK), lambda j: (0, 0)),
            pl.BlockSpec((q, TOPK), lambda j: (0, 0)),
            pl.BlockSpec((q, 128), lambda j: (0, 0)),
        ],
        out_shape=[
            jax.ShapeDtypeStruct((q, TOPK), jnp.float32),
            jax.ShapeDtypeStruct((q, TOPK), jnp.int32),
            jax.ShapeDtypeStruct((q, 128), jnp.int32),
        ],
        scratch_shapes=[
            pltpu.VMEM((q, 128), jnp.float32),
            pltpu.VMEM((q, 128), jnp.int32),
            pltpu.VMEM((q, nkb * 128), jnp.float32),
        ],
    )(queries, keys)


def _body_full(nkeys, nkb, q_ref, k_ref, pv_ref, pi_ref, rv_scr, ri_scr):
    j = pl.program_id(0)

    @pl.when(j == 0)
    def _init():
        rv_scr[...] = jnp.full(rv_scr.shape, -jnp.inf, jnp.float32)
        ri_scr[...] = jnp.full(ri_scr.shape, -1, jnp.int32)

    s = jax.lax.dot_general(
        q_ref[...], k_ref[...], (((1,), (1,)), ((), ())),
        preferred_element_type=jnp.float32)
    col = jax.lax.broadcasted_iota(jnp.int32, s.shape, 1) + j * KB
    s = jnp.where(col < nkeys, s, -jnp.inf)
    tv, ti = _extract_topk(s, col, TOPK)
    mv = jnp.concatenate([rv_scr[...], tv], axis=1)
    mi = jnp.concatenate([ri_scr[...], ti], axis=1)
    fv, fi = _extract_topk(mv, mi, TOPK)
    rv_scr[...] = fv
    ri_scr[...] = fi

    @pl.when(j == nkb - 1)
    def _finalize():
        e = jnp.exp(fv - fv[:, :1])
        pv_ref[...] = e / jnp.sum(e, axis=1, keepdims=True)
        pi_ref[...] = fi


def _full(queries, keys):
    q, d = queries.shape
    n, _ = keys.shape
    nkb = pl.cdiv(n, KB)
    return pl.pallas_call(
        functools.partial(_body_full, n, nkb),
        grid=(nkb,),
        in_specs=[
            pl.BlockSpec((q, d), lambda j: (0, 0)),
            pl.BlockSpec((KB, d), lambda j: (j, 0)),
        ],
        out_specs=[
            pl.BlockSpec((q, TOPK), lambda j: (0, 0)),
            pl.BlockSpec((q, TOPK), lambda j: (0, 0)),
        ],
        out_shape=[
            jax.ShapeDtypeStruct((q, TOPK), jnp.float32),
            jax.ShapeDtypeStruct((q, TOPK), jnp.int32),
        ],
        scratch_shapes=[
            pltpu.VMEM((q, TOPK), jnp.float32),
            pltpu.VMEM((q, TOPK), jnp.int32),
        ],
    )(queries, keys)


def kernel(queries, keys):
    pv, pi, flag = _fast(queries, keys)
    return jax.lax.cond(
        jnp.any(flag != 0),
        lambda: _full(queries, keys),
        lambda: (pv, pi),
    )


# speedup vs baseline: 6.6909x; 1.6800x over previous
import functools

import jax
import jax.numpy as jnp
from jax.experimental import pallas as pl
from jax.experimental.pallas import tpu as pltpu
from jax.experimental.pallas import tpu_sc as plsc

TOPK = 20
KB = 2048
NSL = KB // 128

_INT_MAX = 2147483647


def _extract_topk(v, idx, k):
    vs, ids = [], []
    for _ in range(k):
        m = jnp.max(v, axis=1, keepdims=True)
        pick = jnp.min(jnp.where(v == m, idx, _INT_MAX), axis=1, keepdims=True)
        vs.append(m)
        ids.append(pick)
        v = jnp.where(idx == pick, -jnp.inf, v)
    return jnp.concatenate(vs, axis=1), jnp.concatenate(ids, axis=1)


def _body_scores(nkeys, q_ref, k_ref, s_out, gm_out):
    j = pl.program_id(0)
    s = jax.lax.dot_general(
        q_ref[...], k_ref[...], (((1,), (1,)), ((), ())),
        preferred_element_type=jnp.float32)
    col = jax.lax.broadcasted_iota(jnp.int32, s.shape, 1) + j * KB
    s = jnp.where(col < nkeys, s, -jnp.inf)
    s_out[...] = s
    lane = jax.lax.broadcasted_iota(jnp.int32, (s.shape[0], 128), 1)
    acc = jnp.full((s.shape[0], 128), -jnp.inf, jnp.float32)
    for a in range(NSL):
        r = jnp.max(s[:, a * 128:(a + 1) * 128], axis=1, keepdims=True)
        acc = jnp.where(lane == a, r, acc)
    gm_out[...] = acc


def _scores_and_groupmax(queries, keys):
    q, d = queries.shape
    n, _ = keys.shape
    nkb = pl.cdiv(n, KB)
    return pl.pallas_call(
        functools.partial(_body_scores, n),
        grid=(nkb,),
        in_specs=[
            pl.BlockSpec((q, d), lambda j: (0, 0)),
            pl.BlockSpec((KB, d), lambda j: (j, 0)),
        ],
        out_specs=[
            pl.BlockSpec((q, KB), lambda j: (0, j)),
            pl.BlockSpec((q, 128), lambda j: (0, j)),
        ],
        out_shape=[
            jax.ShapeDtypeStruct((q, nkb * KB), jnp.float32),
            jax.ShapeDtypeStruct((q, nkb * 128), jnp.float32),
        ],
    )(queries, keys)


def _body_groups(ngroups, q_ref_gm, rows_ref, fl_ref):
    qb = pl.program_id(0)
    gm = q_ref_gm[...]
    c = jax.lax.broadcasted_iota(jnp.int32, gm.shape, 1)
    tile = c >> 7
    l = c & 127
    gid = jnp.where(l < NSL, tile * NSL + l, _INT_MAX)
    tv, tg = _extract_topk(gm, gid, TOPK + 1)
    bad = (tv[:, TOPK:TOPK + 1] == tv[:, TOPK - 1:TOPK])
    fl_ref[...] = jnp.broadcast_to(bad, fl_ref.shape).astype(jnp.int32)
    qrow = (jax.lax.broadcasted_iota(jnp.int32, (gm.shape[0], TOPK), 0)
            + qb * gm.shape[0])
    rows_ref[...] = qrow * ngroups + tg[:, :TOPK]


def _group_select(gm, ngroups):
    q = gm.shape[0]
    qblk = min(128, q)
    return pl.pallas_call(
        functools.partial(_body_groups, ngroups),
        grid=(q // qblk,),
        in_specs=[pl.BlockSpec((qblk, gm.shape[1]), lambda i: (i, 0))],
        out_specs=[
            pl.BlockSpec((qblk, TOPK), lambda i: (i, 0)),
            pl.BlockSpec((qblk, 128), lambda i: (i, 0)),
        ],
        out_shape=[
            jax.ShapeDtypeStruct((q, TOPK), jnp.int32),
            jax.ShapeDtypeStruct((q, 128), jnp.int32),
        ],
    )(gm)


_GATHER_WINDOW = 128


def _sc_gather(s_rows, rows):
    nidx = rows.shape[0] * rows.shape[1]
    idx = rows.reshape(1, nidx)
    mesh = plsc.VectorSubcoreMesh(core_axis_name="core",
                                  subcore_axis_name="subcore")

    @pl.kernel(out_type=jax.ShapeDtypeStruct((nidx, 128), jnp.float32),
               mesh=mesh)
    def kern(x_hbm, i_hbm, o_hbm):
        def body(i_vmem, o_vmem):
            pltpu.sync_copy(x_hbm.at[i_vmem.at[0]], o_vmem)

        pltpu.emit_pipeline(
            body,
            grid=(nidx // _GATHER_WINDOW,),
            in_specs=[pl.BlockSpec((1, _GATHER_WINDOW),
                                   index_map=lambda i: (0, i))],
            out_specs=[pl.BlockSpec((_GATHER_WINDOW, 128),
                                    index_map=lambda i: (i, 0))],
            core_axis_name='subcore',
            dimension_semantics=(pltpu.PARALLEL,),
        )(i_hbm, o_hbm)

    return kern(s_rows, idx)


def _body_final(ngroups, cand_ref, rows_ref, pv_ref, pi_ref):
    qb = pl.program_id(0)
    cand = cand_ref[...]
    rows = rows_ref[...]
    qrow = (jax.lax.broadcasted_iota(jnp.int32, (cand.shape[0], 1), 0)
            + qb * cand.shape[0])
    lane = jax.lax.broadcasted_iota(jnp.int32, (cand.shape[0], 128), 1)
    kidx = jnp.concatenate(
        [(rows[:, r:r + 1] - qrow * ngroups) * 128 + lane
         for r in range(TOPK)], axis=1)
    fv, fi = _extract_topk(cand, kidx, TOPK)
    e = jnp.exp(fv - fv[:, :1])
    pv_ref[...] = e / jnp.sum(e, axis=1, keepdims=True)
    pi_ref[...] = fi


def _final_topk(cand, rows, ngroups):
    q = cand.shape[0]
    qblk = min(128, q)
    return pl.pallas_call(
        functools.partial(_body_final, ngroups),
        grid=(q // qblk,),
        in_specs=[
            pl.BlockSpec((qblk, cand.shape[1]), lambda i: (i, 0)),
            pl.BlockSpec((qblk, TOPK), lambda i: (i, 0)),
        ],
        out_specs=[
            pl.BlockSpec((qblk, TOPK), lambda i: (i, 0)),
            pl.BlockSpec((qblk, TOPK), lambda i: (i, 0)),
        ],
        out_shape=[
            jax.ShapeDtypeStruct((q, TOPK), jnp.float32),
            jax.ShapeDtypeStruct((q, TOPK), jnp.int32),
        ],
    )(cand, rows)


def _body_full(nkeys, nkb, q_ref, k_ref, pv_ref, pi_ref, rv_scr, ri_scr):
    j = pl.program_id(0)

    @pl.when(j == 0)
    def _init():
        rv_scr[...] = jnp.full(rv_scr.shape, -jnp.inf, jnp.float32)
        ri_scr[...] = jnp.full(ri_scr.shape, -1, jnp.int32)

    s = jax.lax.dot_general(
        q_ref[...], k_ref[...], (((1,), (1,)), ((), ())),
        preferred_element_type=jnp.float32)
    col = jax.lax.broadcasted_iota(jnp.int32, s.shape, 1) + j * KB
    s = jnp.where(col < nkeys, s, -jnp.inf)
    tv, ti = _extract_topk(s, col, TOPK)
    mv = jnp.concatenate([rv_scr[...], tv], axis=1)
    mi = jnp.concatenate([ri_scr[...], ti], axis=1)
    fv, fi = _extract_topk(mv, mi, TOPK)
    rv_scr[...] = fv
    ri_scr[...] = fi

    @pl.when(j == nkb - 1)
    def _finalize():
        e = jnp.exp(fv - fv[:, :1])
        pv_ref[...] = e / jnp.sum(e, axis=1, keepdims=True)
        pi_ref[...] = fi


def _full(queries, keys):
    q, d = queries.shape
    n, _ = keys.shape
    nkb = pl.cdiv(n, KB)
    return pl.pallas_call(
        functools.partial(_body_full, n, nkb),
        grid=(nkb,),
        in_specs=[
            pl.BlockSpec((q, d), lambda j: (0, 0)),
            pl.BlockSpec((KB, d), lambda j: (j, 0)),
        ],
        out_specs=[
            pl.BlockSpec((q, TOPK), lambda j: (0, 0)),
            pl.BlockSpec((q, TOPK), lambda j: (0, 0)),
        ],
        out_shape=[
            jax.ShapeDtypeStruct((q, TOPK), jnp.float32),
            jax.ShapeDtypeStruct((q, TOPK), jnp.int32),
        ],
        scratch_shapes=[
            pltpu.VMEM((q, TOPK), jnp.float32),
            pltpu.VMEM((q, TOPK), jnp.int32),
        ],
    )(queries, keys)


def kernel(queries, keys):
    q, _ = queries.shape
    n, _ = keys.shape
    nkb = pl.cdiv(n, KB)
    ngroups = nkb * NSL
    s, gm = _scores_and_groupmax(queries, keys)
    rows, flag = _group_select(gm, ngroups)
    gathered = _sc_gather(s.reshape(q * ngroups, 128), rows)
    pv, pi = _final_topk(gathered.reshape(q, TOPK * 128), rows, ngroups)
    return jax.lax.cond(
        jnp.any(flag != 0),
        lambda: _full(queries, keys),
        lambda: (pv, pi),
    )


# speedup vs baseline: 7.1855x; 1.0739x over previous
import functools

import jax
import jax.numpy as jnp
from jax.experimental import pallas as pl
from jax.experimental.pallas import tpu as pltpu
from jax.experimental.pallas import tpu_sc as plsc

TOPK = 20
KB = 2048
NSL = KB // 128
_GM_PACK = 128 // NSL

_INT_MAX = 2147483647


def _extract_topk(v, idx, k):
    vs, ids = [], []
    for _ in range(k):
        m = jnp.max(v, axis=1, keepdims=True)
        pick = jnp.min(jnp.where(v == m, idx, _INT_MAX), axis=1, keepdims=True)
        vs.append(m)
        ids.append(pick)
        v = jnp.where(idx == pick, -jnp.inf, v)
    return jnp.concatenate(vs, axis=1), jnp.concatenate(ids, axis=1)


def _body_scores(nkeys, q_ref, k_ref, s_out, gm_out):
    j = pl.program_id(0)
    s = jax.lax.dot_general(
        q_ref[...], k_ref[...], (((1,), (1,)), ((), ())),
        preferred_element_type=jnp.float32)
    col = jax.lax.broadcasted_iota(jnp.int32, s.shape, 1) + j * KB
    s = jnp.where(col < nkeys, s, -jnp.inf)
    s_out[...] = s

    @pl.when(j % _GM_PACK == 0)
    def _init_gm():
        gm_out[...] = jnp.full(gm_out.shape, -jnp.inf, jnp.float32)

    sub = (j % _GM_PACK) * NSL
    lane = jax.lax.broadcasted_iota(jnp.int32, (s.shape[0], 128), 1)
    acc = gm_out[...]
    for a in range(NSL):
        r = jnp.max(s[:, a * 128:(a + 1) * 128], axis=1, keepdims=True)
        acc = jnp.where(lane == sub + a, r, acc)
    gm_out[...] = acc


def _scores_and_groupmax(queries, keys):
    q, d = queries.shape
    n, _ = keys.shape
    nkb = pl.cdiv(n, KB)
    return pl.pallas_call(
        functools.partial(_body_scores, n),
        grid=(nkb,),
        in_specs=[
            pl.BlockSpec((q, d), lambda j: (0, 0)),
            pl.BlockSpec((KB, d), lambda j: (j, 0)),
        ],
        out_specs=[
            pl.BlockSpec((q, KB), lambda j: (0, j)),
            pl.BlockSpec((q, 128), lambda j: (0, j // _GM_PACK)),
        ],
        out_shape=[
            jax.ShapeDtypeStruct((q, nkb * KB), jnp.float32),
            jax.ShapeDtypeStruct((q, pl.cdiv(nkb, _GM_PACK) * 128),
                                 jnp.float32),
        ],
    )(queries, keys)


def _body_groups(ngroups, q_ref_gm, rows_ref, fl_ref):
    qb = pl.program_id(0)
    gm = q_ref_gm[...]
    gid = jax.lax.broadcasted_iota(jnp.int32, gm.shape, 1)
    tv, tg = _extract_topk(gm, gid, TOPK + 1)
    bad = (tv[:, TOPK:TOPK + 1] == tv[:, TOPK - 1:TOPK])
    fl_ref[...] = jnp.broadcast_to(bad, fl_ref.shape).astype(jnp.int32)
    qrow = (jax.lax.broadcasted_iota(jnp.int32, (gm.shape[0], TOPK), 0)
            + qb * gm.shape[0])
    rows_ref[...] = qrow * ngroups + tg[:, :TOPK]


def _group_select(gm, ngroups):
    q = gm.shape[0]
    qblk = min(128, q)
    return pl.pallas_call(
        functools.partial(_body_groups, ngroups),
        grid=(q // qblk,),
        in_specs=[pl.BlockSpec((qblk, gm.shape[1]), lambda i: (i, 0))],
        out_specs=[
            pl.BlockSpec((qblk, TOPK), lambda i: (i, 0)),
            pl.BlockSpec((qblk, 128), lambda i: (i, 0)),
        ],
        out_shape=[
            jax.ShapeDtypeStruct((q, TOPK), jnp.int32),
            jax.ShapeDtypeStruct((q, 128), jnp.int32),
        ],
    )(gm)


_GATHER_WINDOW = 128


def _sc_gather(s_rows, rows):
    nidx = rows.shape[0] * rows.shape[1]
    idx = rows.reshape(1, nidx)
    mesh = plsc.VectorSubcoreMesh(core_axis_name="core",
                                  subcore_axis_name="subcore")

    @pl.kernel(out_type=jax.ShapeDtypeStruct((nidx, 128), jnp.float32),
               mesh=mesh)
    def kern(x_hbm, i_hbm, o_hbm):
        def body(i_vmem, o_vmem):
            pltpu.sync_copy(x_hbm.at[i_vmem.at[0]], o_vmem)

        pltpu.emit_pipeline(
            body,
            grid=(nidx // _GATHER_WINDOW,),
            in_specs=[pl.BlockSpec((1, _GATHER_WINDOW),
                                   index_map=lambda i: (0, i))],
            out_specs=[pl.BlockSpec((_GATHER_WINDOW, 128),
                                    index_map=lambda i: (i, 0))],
            core_axis_name='subcore',
            dimension_semantics=(pltpu.PARALLEL,),
        )(i_hbm, o_hbm)

    return kern(s_rows, idx)


def _body_final(ngroups, cand_ref, rows_ref, pv_ref, pi_ref):
    qb = pl.program_id(0)
    cand = cand_ref[...]
    rows = rows_ref[...]
    qrow = (jax.lax.broadcasted_iota(jnp.int32, (cand.shape[0], 1), 0)
            + qb * cand.shape[0])
    lane = jax.lax.broadcasted_iota(jnp.int32, (cand.shape[0], 128), 1)
    kidx = jnp.concatenate(
        [(rows[:, r:r + 1] - qrow * ngroups) * 128 + lane
         for r in range(TOPK)], axis=1)
    fv, fi = _extract_topk(cand, kidx, TOPK)
    e = jnp.exp(fv - fv[:, :1])
    pv_ref[...] = e / jnp.sum(e, axis=1, keepdims=True)
    pi_ref[...] = fi


def _final_topk(cand, rows, ngroups):
    q = cand.shape[0]
    qblk = min(128, q)
    return pl.pallas_call(
        functools.partial(_body_final, ngroups),
        grid=(q // qblk,),
        in_specs=[
            pl.BlockSpec((qblk, cand.shape[1]), lambda i: (i, 0)),
            pl.BlockSpec((qblk, TOPK), lambda i: (i, 0)),
        ],
        out_specs=[
            pl.BlockSpec((qblk, TOPK), lambda i: (i, 0)),
            pl.BlockSpec((qblk, TOPK), lambda i: (i, 0)),
        ],
        out_shape=[
            jax.ShapeDtypeStruct((q, TOPK), jnp.float32),
            jax.ShapeDtypeStruct((q, TOPK), jnp.int32),
        ],
    )(cand, rows)


def _body_full(nkeys, nkb, q_ref, k_ref, pv_ref, pi_ref, rv_scr, ri_scr):
    j = pl.program_id(0)

    @pl.when(j == 0)
    def _init():
        rv_scr[...] = jnp.full(rv_scr.shape, -jnp.inf, jnp.float32)
        ri_scr[...] = jnp.full(ri_scr.shape, -1, jnp.int32)

    s = jax.lax.dot_general(
        q_ref[...], k_ref[...], (((1,), (1,)), ((), ())),
        preferred_element_type=jnp.float32)
    col = jax.lax.broadcasted_iota(jnp.int32, s.shape, 1) + j * KB
    s = jnp.where(col < nkeys, s, -jnp.inf)
    tv, ti = _extract_topk(s, col, TOPK)
    mv = jnp.concatenate([rv_scr[...], tv], axis=1)
    mi = jnp.concatenate([ri_scr[...], ti], axis=1)
    fv, fi = _extract_topk(mv, mi, TOPK)
    rv_scr[...] = fv
    ri_scr[...] = fi

    @pl.when(j == nkb - 1)
    def _finalize():
        e = jnp.exp(fv - fv[:, :1])
        pv_ref[...] = e / jnp.sum(e, axis=1, keepdims=True)
        pi_ref[...] = fi


def _full(queries, keys):
    q, d = queries.shape
    n, _ = keys.shape
    nkb = pl.cdiv(n, KB)
    return pl.pallas_call(
        functools.partial(_body_full, n, nkb),
        grid=(nkb,),
        in_specs=[
            pl.BlockSpec((q, d), lambda j: (0, 0)),
            pl.BlockSpec((KB, d), lambda j: (j, 0)),
        ],
        out_specs=[
            pl.BlockSpec((q, TOPK), lambda j: (0, 0)),
            pl.BlockSpec((q, TOPK), lambda j: (0, 0)),
        ],
        out_shape=[
            jax.ShapeDtypeStruct((q, TOPK), jnp.float32),
            jax.ShapeDtypeStruct((q, TOPK), jnp.int32),
        ],
        scratch_shapes=[
            pltpu.VMEM((q, TOPK), jnp.float32),
            pltpu.VMEM((q, TOPK), jnp.int32),
        ],
    )(queries, keys)


def kernel(queries, keys):
    q, _ = queries.shape
    n, _ = keys.shape
    nkb = pl.cdiv(n, KB)
    ngroups = nkb * NSL
    s, gm = _scores_and_groupmax(queries, keys)
    rows, flag = _group_select(gm, ngroups)
    gathered = _sc_gather(s.reshape(q * ngroups, 128), rows)
    pv, pi = _final_topk(gathered.reshape(q, TOPK * 128), rows, ngroups)
    return jax.lax.cond(
        jnp.any(flag != 0),
        lambda: _full(queries, keys),
        lambda: (pv, pi),
    )


# speedup vs baseline: 7.9542x; 1.1070x over previous
import functools

import jax
import jax.numpy as jnp
from jax.experimental import pallas as pl
from jax.experimental.pallas import tpu as pltpu
from jax.experimental.pallas import tpu_sc as plsc

TOPK = 20
KB = 2048
NSL = KB // 128
_GM_PACK = 128 // NSL

_INT_MAX = 2147483647


def _extract_topk(v, idx, k):
    vs, ids = [], []
    for _ in range(k):
        m = jnp.max(v, axis=1, keepdims=True)
        pick = jnp.min(jnp.where(v == m, idx, _INT_MAX), axis=1, keepdims=True)
        vs.append(m)
        ids.append(pick)
        v = jnp.where(idx == pick, -jnp.inf, v)
    return jnp.concatenate(vs, axis=1), jnp.concatenate(ids, axis=1)


def _body_scores(nkeys, nkb, ngroups, q_ref, k_ref, s_out, rows_ref,
                 fl_ref, gm_scr):
    j = pl.program_id(0)
    s = jax.lax.dot_general(
        q_ref[...], k_ref[...], (((1,), (1,)), ((), ())),
        preferred_element_type=jnp.float32)
    col = jax.lax.broadcasted_iota(jnp.int32, s.shape, 1) + j * KB
    s = jnp.where(col < nkeys, s, -jnp.inf)
    s_out[...] = s

    nq = s.shape[0]
    base = (j // _GM_PACK) * 128
    sub = (j % _GM_PACK) * NSL
    lane = jax.lax.broadcasted_iota(jnp.int32, (nq, 128), 1)
    acc = jnp.full((nq, 128), -jnp.inf, jnp.float32)
    for a in range(NSL):
        r = jnp.max(s[:, a * 128:(a + 1) * 128], axis=1, keepdims=True)
        acc = jnp.where(lane == sub + a, r, acc)

    @pl.when(j % _GM_PACK == 0)
    def _first_tile_of_block():
        gm_scr[:, pl.ds(base, 128)] = acc

    @pl.when(j % _GM_PACK != 0)
    def _other_tiles():
        gm_scr[:, pl.ds(base, 128)] = jnp.maximum(
            gm_scr[:, pl.ds(base, 128)], acc)

    @pl.when(j == nkb - 1)
    def _select_groups():
        gm = gm_scr[...]
        gid = jax.lax.broadcasted_iota(jnp.int32, gm.shape, 1)
        tv, tg = _extract_topk(gm, gid, TOPK + 1)
        bad = (tv[:, TOPK:TOPK + 1] == tv[:, TOPK - 1:TOPK])
        fl_ref[...] = jnp.broadcast_to(bad, fl_ref.shape).astype(jnp.int32)
        qrow = jax.lax.broadcasted_iota(jnp.int32, (nq, TOPK), 0)
        rows_ref[...] = qrow * ngroups + tg[:, :TOPK]


def _scores_and_groups(queries, keys):
    q, d = queries.shape
    n, _ = keys.shape
    nkb = pl.cdiv(n, KB)
    ngroups = nkb * NSL
    ngm = pl.cdiv(nkb, _GM_PACK) * 128
    return pl.pallas_call(
        functools.partial(_body_scores, n, nkb, ngroups),
        grid=(nkb,),
        in_specs=[
            pl.BlockSpec((q, d), lambda j: (0, 0)),
            pl.BlockSpec((KB, d), lambda j: (j, 0)),
        ],
        out_specs=[
            pl.BlockSpec((q, KB), lambda j: (0, j)),
            pl.BlockSpec((q, TOPK), lambda j: (0, 0)),
            pl.BlockSpec((q, 128), lambda j: (0, 0)),
        ],
        out_shape=[
            jax.ShapeDtypeStruct((q, nkb * KB), jnp.float32),
            jax.ShapeDtypeStruct((q, TOPK), jnp.int32),
            jax.ShapeDtypeStruct((q, 128), jnp.int32),
        ],
        scratch_shapes=[
            pltpu.VMEM((q, ngm), jnp.float32),
        ],
    )(queries, keys)


def _body_groups(ngroups, q_ref_gm, rows_ref, fl_ref):
    qb = pl.program_id(0)
    gm = q_ref_gm[...]
    gid = jax.lax.broadcasted_iota(jnp.int32, gm.shape, 1)
    tv, tg = _extract_topk(gm, gid, TOPK + 1)
    bad = (tv[:, TOPK:TOPK + 1] == tv[:, TOPK - 1:TOPK])
    fl_ref[...] = jnp.broadcast_to(bad, fl_ref.shape).astype(jnp.int32)
    qrow = (jax.lax.broadcasted_iota(jnp.int32, (gm.shape[0], TOPK), 0)
            + qb * gm.shape[0])
    rows_ref[...] = qrow * ngroups + tg[:, :TOPK]


def _group_select(gm, ngroups):
    q = gm.shape[0]
    qblk = min(128, q)
    return pl.pallas_call(
        functools.partial(_body_groups, ngroups),
        grid=(q // qblk,),
        in_specs=[pl.BlockSpec((qblk, gm.shape[1]), lambda i: (i, 0))],
        out_specs=[
            pl.BlockSpec((qblk, TOPK), lambda i: (i, 0)),
            pl.BlockSpec((qblk, 128), lambda i: (i, 0)),
        ],
        out_shape=[
            jax.ShapeDtypeStruct((q, TOPK), jnp.int32),
            jax.ShapeDtypeStruct((q, 128), jnp.int32),
        ],
    )(gm)


_GATHER_WINDOW = 128


def _sc_gather(s_rows, rows):
    nidx = rows.shape[0] * rows.shape[1]
    idx = rows.reshape(1, nidx)
    mesh = plsc.VectorSubcoreMesh(core_axis_name="core",
                                  subcore_axis_name="subcore")

    @pl.kernel(out_type=jax.ShapeDtypeStruct((nidx, 128), jnp.float32),
               mesh=mesh)
    def kern(x_hbm, i_hbm, o_hbm):
        def body(i_vmem, o_vmem):
            pltpu.sync_copy(x_hbm.at[i_vmem.at[0]], o_vmem)

        pltpu.emit_pipeline(
            body,
            grid=(nidx // _GATHER_WINDOW,),
            in_specs=[pl.BlockSpec((1, _GATHER_WINDOW),
                                   index_map=lambda i: (0, i))],
            out_specs=[pl.BlockSpec((_GATHER_WINDOW, 128),
                                    index_map=lambda i: (i, 0))],
            core_axis_name='subcore',
            dimension_semantics=(pltpu.PARALLEL,),
        )(i_hbm, o_hbm)

    return kern(s_rows, idx)


def _body_final(ngroups, cand_ref, rows_ref, pv_ref, pi_ref):
    qb = pl.program_id(0)
    cand = cand_ref[...]
    rows = rows_ref[...]
    qrow = (jax.lax.broadcasted_iota(jnp.int32, (cand.shape[0], 1), 0)
            + qb * cand.shape[0])
    lane = jax.lax.broadcasted_iota(jnp.int32, (cand.shape[0], 128), 1)
    kidx = jnp.concatenate(
        [(rows[:, r:r + 1] - qrow * ngroups) * 128 + lane
         for r in range(TOPK)], axis=1)
    fv, fi = _extract_topk(cand, kidx, TOPK)
    e = jnp.exp(fv - fv[:, :1])
    pv_ref[...] = e / jnp.sum(e, axis=1, keepdims=True)
    pi_ref[...] = fi


def _final_topk(cand, rows, ngroups):
    q = cand.shape[0]
    qblk = min(128, q)
    return pl.pallas_call(
        functools.partial(_body_final, ngroups),
        grid=(q // qblk,),
        in_specs=[
            pl.BlockSpec((qblk, cand.shape[1]), lambda i: (i, 0)),
            pl.BlockSpec((qblk, TOPK), lambda i: (i, 0)),
        ],
        out_specs=[
            pl.BlockSpec((qblk, TOPK), lambda i: (i, 0)),
            pl.BlockSpec((qblk, TOPK), lambda i: (i, 0)),
        ],
        out_shape=[
            jax.ShapeDtypeStruct((q, TOPK), jnp.float32),
            jax.ShapeDtypeStruct((q, TOPK), jnp.int32),
        ],
    )(cand, rows)


def _body_full(nkeys, nkb, q_ref, k_ref, pv_ref, pi_ref, rv_scr, ri_scr):
    j = pl.program_id(0)

    @pl.when(j == 0)
    def _init():
        rv_scr[...] = jnp.full(rv_scr.shape, -jnp.inf, jnp.float32)
        ri_scr[...] = jnp.full(ri_scr.shape, -1, jnp.int32)

    s = jax.lax.dot_general(
        q_ref[...], k_ref[...], (((1,), (1,)), ((), ())),
        preferred_element_type=jnp.float32)
    col = jax.lax.broadcasted_iota(jnp.int32, s.shape, 1) + j * KB
    s = jnp.where(col < nkeys, s, -jnp.inf)
    tv, ti = _extract_topk(s, col, TOPK)
    mv = jnp.concatenate([rv_scr[...], tv], axis=1)
    mi = jnp.concatenate([ri_scr[...], ti], axis=1)
    fv, fi = _extract_topk(mv, mi, TOPK)
    rv_scr[...] = fv
    ri_scr[...] = fi

    @pl.when(j == nkb - 1)
    def _finalize():
        e = jnp.exp(fv - fv[:, :1])
        pv_ref[...] = e / jnp.sum(e, axis=1, keepdims=True)
        pi_ref[...] = fi


def _full(queries, keys):
    q, d = queries.shape
    n, _ = keys.shape
    nkb = pl.cdiv(n, KB)
    return pl.pallas_call(
        functools.partial(_body_full, n, nkb),
        grid=(nkb,),
        in_specs=[
            pl.BlockSpec((q, d), lambda j: (0, 0)),
            pl.BlockSpec((KB, d), lambda j: (j, 0)),
        ],
        out_specs=[
            pl.BlockSpec((q, TOPK), lambda j: (0, 0)),
            pl.BlockSpec((q, TOPK), lambda j: (0, 0)),
        ],
        out_shape=[
            jax.ShapeDtypeStruct((q, TOPK), jnp.float32),
            jax.ShapeDtypeStruct((q, TOPK), jnp.int32),
        ],
        scratch_shapes=[
            pltpu.VMEM((q, TOPK), jnp.float32),
            pltpu.VMEM((q, TOPK), jnp.int32),
        ],
    )(queries, keys)


def kernel(queries, keys):
    q, _ = queries.shape
    n, _ = keys.shape
    nkb = pl.cdiv(n, KB)
    ngroups = nkb * NSL
    s, rows, flag = _scores_and_groups(queries, keys)
    gathered = _sc_gather(s.reshape(q * ngroups, 128), rows)
    pv, pi = _final_topk(gathered.reshape(q, TOPK * 128), rows, ngroups)
    return jax.lax.cond(
        jnp.any(flag != 0),
        lambda: _full(queries, keys),
        lambda: (pv, pi),
    )
